# baseline (device time: 101931 ns/iter reference)
import jax
import jax.numpy as jnp
from jax import lax
from jax.experimental import pallas as pl
from jax.experimental.pallas import tpu as pltpu

B, S, D = 1, 1024, 2048
DC = 128
H, Dh, Dr = 16, 128, 32
HG = 4
NG = H // HG
SCALE = (Dh + Dr) ** -0.5
F32 = jnp.float32
BF16 = jnp.bfloat16


def _kv_exchange(x2, Wdkv, Wuk, Wuv, Wkr):

    def body(x_ref, wdkv_ref, wuk_ref, wuv_ref, wkr_ref,
             k_ref, v_ref, kr_ref,
             c_ref, cr_ref, wukb_ref, wuvb_ref, wukr_ref, wuvr_ref,
             send_sems, recv_sems):
        my_x = lax.axis_index("x")
        my_y = lax.axis_index("y")
        my_z = lax.axis_index("z")
        partner = (my_x, 1 - my_y, my_z)

        barrier_sem = pltpu.get_barrier_semaphore()
        pl.semaphore_signal(barrier_sem, inc=1, device_id=partner,
                            device_id_type=pl.DeviceIdType.MESH)
        pl.semaphore_wait(barrier_sem, 1)

        c32 = jnp.dot(x_ref[...], wdkv_ref[...], preferred_element_type=F32)
        c_ref[...] = c32.astype(BF16)
        wukb_ref[...] = wuk_ref[...].astype(BF16)
        wuvb_ref[...] = wuv_ref[...].astype(BF16)

        rdmas = []
        for i, (src, dst) in enumerate(
            [(c_ref, cr_ref), (wukb_ref, wukr_ref), (wuvb_ref, wuvr_ref)]
        ):
            rdma = pltpu.make_async_remote_copy(
                src_ref=src, dst_ref=dst,
                send_sem=send_sems.at[i], recv_sem=recv_sems.at[i],
                device_id=partner, device_id_type=pl.DeviceIdType.MESH,
            )
            rdma.start()
            rdmas.append(rdma)

        kr_ref[...] = jnp.dot(x_ref[...], wkr_ref[...],
                              preferred_element_type=F32)
        k32 = jnp.dot(c32, wuk_ref[...], preferred_element_type=F32)
        v32 = jnp.dot(c32, wuv_ref[...], preferred_element_type=F32)

        for rdma in rdmas:
            rdma.wait()

        k32 += jnp.dot(cr_ref[...], wukr_ref[...], preferred_element_type=F32)
        v32 += jnp.dot(cr_ref[...], wuvr_ref[...], preferred_element_type=F32)
        k_ref[...] = k32.astype(BF16)
        v_ref[...] = v32.astype(BF16)

    return pl.pallas_call(
        body,
        out_shape=[
            jax.ShapeDtypeStruct((S, D), BF16),
            jax.ShapeDtypeStruct((S, D), BF16),
            jax.ShapeDtypeStruct((S, Dr), F32),
        ],
        in_specs=[pl.BlockSpec(memory_space=pltpu.VMEM)] * 5,
        out_specs=[pl.BlockSpec(memory_space=pltpu.VMEM)] * 3,
        scratch_shapes=[
            pltpu.VMEM((S, DC), BF16),
            pltpu.VMEM((S, DC), BF16),
            pltpu.VMEM((DC, D), BF16),
            pltpu.VMEM((DC, D), BF16),
            pltpu.VMEM((DC, D), BF16),
            pltpu.VMEM((DC, D), BF16),
            pltpu.SemaphoreType.DMA((3,)),
            pltpu.SemaphoreType.DMA((3,)),
        ],
        compiler_params=pltpu.CompilerParams(collective_id=0),
    )(x2, Wdkv, Wuk, Wuv, Wkr)


def _attn_out(K, V, x2, Wq, Wqr, Kr, Wo):

    def body(ys_ref, k_ref, v_ref, x_ref, wq_ref, wqr_ref, kr_ref,
             wo_ref, out_ref, og_send, og_recv, send_sems, recv_sems):
        hp = pl.program_id(0)
        my_x = lax.axis_index("x")
        my_y = lax.axis_index("y")
        my_z = lax.axis_index("z")
        partner = (my_x, 1 - my_y, my_z)

        @pl.when(hp == 0)
        def _():
            barrier_sem = pltpu.get_barrier_semaphore()
            pl.semaphore_signal(barrier_sem, inc=1, device_id=partner,
                                device_id_type=pl.DeviceIdType.MESH)
            pl.semaphore_wait(barrier_sem, 1)

        wob = wo_ref[...].astype(BF16)

        @pl.when(hp < 2)
        def _():
            qg = jnp.dot(x_ref[...], wq_ref[...],
                         preferred_element_type=F32)
            qrg = jnp.dot(x_ref[...], wqr_ref[...],
                          preferred_element_type=F32)
            kr = kr_ref[...].astype(BF16)
            ones = jnp.ones((S, Dh), BF16)
            o_parts = []
            for j in range(HG):
                q = qg[:, j * Dh:(j + 1) * Dh]
                qr = qrg[:, j * Dr:(j + 1) * Dr]
                k = k_ref[:, j * Dh:(j + 1) * Dh]
                v = v_ref[:, j * Dh:(j + 1) * Dh]
                qa = (jnp.concatenate([q, qr], axis=1) * SCALE).astype(BF16)
                ka = jnp.concatenate([k, kr], axis=1)
                s = lax.dot_general(qa, ka, (((1,), (1,)), ((), ())),
                                    preferred_element_type=F32)
                p = jnp.exp(s)
                va = jnp.concatenate([v, ones], axis=1)
                o2 = jnp.dot(p, va, preferred_element_type=F32)
                o = o2[:, :Dh] * (1.0 / o2[:, Dh:Dh + 1])
                o_parts.append(o)
            og_lo = jnp.concatenate(o_parts[:2], axis=1).astype(BF16)
            og_hi = jnp.concatenate(o_parts[2:], axis=1).astype(BF16)
            contrib = (
                jnp.dot(og_lo, wob[:HG * Dh // 2, :],
                        preferred_element_type=F32)
                + jnp.dot(og_hi, wob[HG * Dh // 2:, :],
                          preferred_element_type=F32)
            )

            @pl.when(hp == 0)
            def _():
                out_ref[...] = contrib

            @pl.when(hp == 1)
            def _():
                out_ref[...] += contrib

            for step in (0, 1):
                @pl.when(hp == step)
                def _(step=step, og_lo=og_lo, og_hi=og_hi):
                    for half, og_half in ((0, og_lo), (1, og_hi)):
                        slot = step * 2 + half
                        og_send[slot, :, :] = og_half
                        rdma = pltpu.make_async_remote_copy(
                            src_ref=og_send.at[slot],
                            dst_ref=og_recv.at[slot],
                            send_sem=send_sems.at[slot],
                            recv_sem=recv_sems.at[slot],
                            device_id=partner,
                            device_id_type=pl.DeviceIdType.MESH,
                        )
                        rdma.start()

        @pl.when(hp >= 2)
        def _():
            for step in (0, 1):
                @pl.when(hp == step + 2)
                def _(step=step):
                    acc = out_ref[...]
                    for half in (0, 1):
                        slot = step * 2 + half
                        rdma = pltpu.make_async_remote_copy(
                            src_ref=og_send.at[slot],
                            dst_ref=og_recv.at[slot],
                            send_sem=send_sems.at[slot],
                            recv_sem=recv_sems.at[slot],
                            device_id=partner,
                            device_id_type=pl.DeviceIdType.MESH,
                        )
                        rdma.wait_recv()
                        lo = half * (HG * Dh // 2)
                        acc += jnp.dot(
                            og_recv[slot, :, :],
                            wob[lo:lo + HG * Dh // 2, :],
                            preferred_element_type=F32)
                    out_ref[...] = acc

            @pl.when(hp == 3)
            def _():
                for slot in range(4):
                    pltpu.make_async_remote_copy(
                        src_ref=og_send.at[slot],
                        dst_ref=og_recv.at[slot],
                        send_sem=send_sems.at[slot],
                        recv_sem=recv_sems.at[slot],
                        device_id=partner,
                        device_id_type=pl.DeviceIdType.MESH,
                    ).wait_send()

    def _mine(h, ys):
        return ys[0] * 2 + jnp.minimum(h, 1)

    def _wo_group(h, ys):
        return jnp.where(h < 2, ys[0] * 2 + h, (1 - ys[0]) * 2 + h - 2)

    grid_spec = pltpu.PrefetchScalarGridSpec(
        num_scalar_prefetch=1,
        grid=(4,),
        in_specs=[
            pl.BlockSpec((S, HG * Dh), lambda h, ys: (0, _mine(h, ys))),
            pl.BlockSpec((S, HG * Dh), lambda h, ys: (0, _mine(h, ys))),
            pl.BlockSpec((S, D), lambda h, ys: (0, 0)),
            pl.BlockSpec((D, HG * Dh), lambda h, ys: (0, _mine(h, ys))),
            pl.BlockSpec((D, HG * Dr), lambda h, ys: (0, jnp.minimum(h, 1))),
            pl.BlockSpec((S, Dr), lambda h, ys: (0, 0)),
            pl.BlockSpec((HG * Dh, D), lambda h, ys: (_wo_group(h, ys), 0)),
        ],
        out_specs=pl.BlockSpec((S, D), lambda h, ys: (0, 0)),
        scratch_shapes=[
            pltpu.VMEM((4, S, HG * Dh // 2), BF16),
            pltpu.VMEM((4, S, HG * Dh // 2), BF16),
            pltpu.SemaphoreType.DMA((4,)),
            pltpu.SemaphoreType.DMA((4,)),
        ],
    )

    y = lax.axis_index("y")
    ys = y.astype(jnp.int32).reshape(1)
    Wqr_my = lax.dynamic_slice(Wqr, (0, y * (2 * HG * Dr)), (D, 2 * HG * Dr))
    return pl.pallas_call(
        body,
        grid_spec=grid_spec,
        out_shape=jax.ShapeDtypeStruct((S, D), F32),
        compiler_params=pltpu.CompilerParams(
            collective_id=1,
            vmem_limit_bytes=100 * 1024 * 1024,
        ),
    )(ys, K, V, x2, Wq, Wqr_my, Kr, Wo)


def kernel(x, Wdkv, Wuk, Wuv, Wq, Wqr, Wkr, Wo):
    x2 = x.reshape(S, D)
    K, V, Kr = _kv_exchange(x2, Wdkv, Wuk, Wuv, Wkr)
    out = _attn_out(K, V, x2, Wq, Wqr, Kr, Wo)
    return out.reshape(B, S, D)


# device time: 98076 ns/iter; 1.0393x vs baseline; 1.0393x over previous
import jax
import jax.numpy as jnp
from jax import lax
from jax.experimental import pallas as pl
from jax.experimental.pallas import tpu as pltpu

B, S, D = 1, 1024, 2048
DC = 128
H, Dh, Dr = 16, 128, 32
HG = 4
NG = H // HG
SCALE = (Dh + Dr) ** -0.5
F32 = jnp.float32
BF16 = jnp.bfloat16


def _kv_exchange(x2, Wdkv, Wuk, Wuv, Wkr):

    def body(x_ref, wdkv_ref, wuk_ref, wuv_ref, wkr_ref,
             k_ref, v_ref, kr_ref,
             c_ref, cr_ref, wukb_ref, wuvb_ref, wukr_ref, wuvr_ref,
             send_sems, recv_sems):
        my_x = lax.axis_index("x")
        my_y = lax.axis_index("y")
        my_z = lax.axis_index("z")
        partner = (my_x, 1 - my_y, my_z)

        barrier_sem = pltpu.get_barrier_semaphore()
        pl.semaphore_signal(barrier_sem, inc=1, device_id=partner,
                            device_id_type=pl.DeviceIdType.MESH)
        pl.semaphore_wait(barrier_sem, 1)

        c32 = jnp.dot(x_ref[...], wdkv_ref[...], preferred_element_type=F32)
        c_ref[...] = c32.astype(BF16)
        wukb_ref[...] = wuk_ref[...].astype(BF16)
        wuvb_ref[...] = wuv_ref[...].astype(BF16)

        rdmas = []
        for i, (src, dst) in enumerate(
            [(c_ref, cr_ref), (wukb_ref, wukr_ref), (wuvb_ref, wuvr_ref)]
        ):
            rdma = pltpu.make_async_remote_copy(
                src_ref=src, dst_ref=dst,
                send_sem=send_sems.at[i], recv_sem=recv_sems.at[i],
                device_id=partner, device_id_type=pl.DeviceIdType.MESH,
            )
            rdma.start()
            rdmas.append(rdma)

        kr_ref[...] = jnp.dot(x_ref[...], wkr_ref[...],
                              preferred_element_type=F32)
        k32 = jnp.dot(c32, wuk_ref[...], preferred_element_type=F32)
        v32 = jnp.dot(c32, wuv_ref[...], preferred_element_type=F32)

        for rdma in rdmas:
            rdma.wait()

        k32 += jnp.dot(cr_ref[...], wukr_ref[...], preferred_element_type=F32)
        v32 += jnp.dot(cr_ref[...], wuvr_ref[...], preferred_element_type=F32)
        k_ref[...] = k32.astype(BF16)
        v_ref[...] = v32.astype(BF16)

    return pl.pallas_call(
        body,
        out_shape=[
            jax.ShapeDtypeStruct((S, D), BF16),
            jax.ShapeDtypeStruct((S, D), BF16),
            jax.ShapeDtypeStruct((S, Dr), F32),
        ],
        in_specs=[pl.BlockSpec(memory_space=pltpu.VMEM)] * 5,
        out_specs=[pl.BlockSpec(memory_space=pltpu.VMEM)] * 3,
        scratch_shapes=[
            pltpu.VMEM((S, DC), BF16),
            pltpu.VMEM((S, DC), BF16),
            pltpu.VMEM((DC, D), BF16),
            pltpu.VMEM((DC, D), BF16),
            pltpu.VMEM((DC, D), BF16),
            pltpu.VMEM((DC, D), BF16),
            pltpu.SemaphoreType.DMA((3,)),
            pltpu.SemaphoreType.DMA((3,)),
        ],
        compiler_params=pltpu.CompilerParams(collective_id=0),
    )(x2, Wdkv, Wuk, Wuv, Wkr)


def _attn_out(K, V, x2, Wq, Wqr, Kr, Wo):

    def body(ys_ref, k_ref, v_ref, x_ref, wq_ref, wqr_ref, kr_ref,
             wo_ref, out_ref, qrg_buf, og_send, og_recv,
             send_sems, recv_sems):
        hp = pl.program_id(0)
        my_x = lax.axis_index("x")
        my_y = lax.axis_index("y")
        my_z = lax.axis_index("z")
        partner = (my_x, 1 - my_y, my_z)

        @pl.when(hp == 0)
        def _():
            barrier_sem = pltpu.get_barrier_semaphore()
            pl.semaphore_signal(barrier_sem, inc=1, device_id=partner,
                                device_id_type=pl.DeviceIdType.MESH)
            pl.semaphore_wait(barrier_sem, 1)

        wob = wo_ref[...].astype(BF16)

        @pl.when(hp < 2)
        def _():
            qg = jnp.dot(x_ref[...], wq_ref[...],
                         preferred_element_type=F32)
            for yv in (0, 1):
                for hh in (0, 1):
                    @pl.when((my_y == yv) & (hp == hh))
                    def _(yv=yv, hh=hh):
                        lo = (yv * 2 + hh) * (HG * Dr)
                        qrg_buf[...] = jnp.dot(
                            x_ref[...], wqr_ref[:, lo:lo + HG * Dr],
                            preferred_element_type=F32)
            qrg = qrg_buf[...]
            kr = kr_ref[...].astype(BF16)
            ones = jnp.ones((S, Dh), BF16)
            o_parts = []
            for j in range(HG):
                q = qg[:, j * Dh:(j + 1) * Dh]
                qr = qrg[:, j * Dr:(j + 1) * Dr]
                k = k_ref[:, j * Dh:(j + 1) * Dh]
                v = v_ref[:, j * Dh:(j + 1) * Dh]
                qa = (jnp.concatenate([q, qr], axis=1) * SCALE).astype(BF16)
                ka = jnp.concatenate([k, kr], axis=1)
                s = lax.dot_general(qa, ka, (((1,), (1,)), ((), ())),
                                    preferred_element_type=F32)
                p = jnp.exp(s)
                va = jnp.concatenate([v, ones], axis=1)
                o2 = jnp.dot(p, va, preferred_element_type=F32)
                o = o2[:, :Dh] * (1.0 / o2[:, Dh:Dh + 1])
                o_parts.append(o)
            og = jnp.concatenate(o_parts, axis=1).astype(BF16)
            contrib = jnp.dot(og, wob, preferred_element_type=F32)

            @pl.when(hp == 0)
            def _():
                out_ref[...] = contrib

            @pl.when(hp == 1)
            def _():
                out_ref[...] += contrib

            for slot in (0, 1):
                @pl.when(hp == slot)
                def _(slot=slot):
                    og_send[slot, :, :] = og
                    rdma = pltpu.make_async_remote_copy(
                        src_ref=og_send.at[slot],
                        dst_ref=og_recv.at[slot],
                        send_sem=send_sems.at[slot],
                        recv_sem=recv_sems.at[slot],
                        device_id=partner,
                        device_id_type=pl.DeviceIdType.MESH,
                    )
                    rdma.start()

        @pl.when(hp >= 2)
        def _():
            for slot in (0, 1):
                @pl.when(hp == slot + 2)
                def _(slot=slot):
                    rdma = pltpu.make_async_remote_copy(
                        src_ref=og_send.at[slot],
                        dst_ref=og_recv.at[slot],
                        send_sem=send_sems.at[slot],
                        recv_sem=recv_sems.at[slot],
                        device_id=partner,
                        device_id_type=pl.DeviceIdType.MESH,
                    )
                    rdma.wait_recv()
                    out_ref[...] += jnp.dot(og_recv[slot, :, :], wob,
                                            preferred_element_type=F32)

            @pl.when(hp == 3)
            def _():
                for slot in (0, 1):
                    pltpu.make_async_remote_copy(
                        src_ref=og_send.at[slot],
                        dst_ref=og_recv.at[slot],
                        send_sem=send_sems.at[slot],
                        recv_sem=recv_sems.at[slot],
                        device_id=partner,
                        device_id_type=pl.DeviceIdType.MESH,
                    ).wait_send()

    def _mine(h, ys):
        return ys[0] * 2 + jnp.minimum(h, 1)

    def _wo_group(h, ys):
        return jnp.where(h < 2, ys[0] * 2 + h, (1 - ys[0]) * 2 + h - 2)

    grid_spec = pltpu.PrefetchScalarGridSpec(
        num_scalar_prefetch=1,
        grid=(4,),
        in_specs=[
            pl.BlockSpec((S, HG * Dh), lambda h, ys: (0, _mine(h, ys))),
            pl.BlockSpec((S, HG * Dh), lambda h, ys: (0, _mine(h, ys))),
            pl.BlockSpec((S, D), lambda h, ys: (0, 0)),
            pl.BlockSpec((D, HG * Dh), lambda h, ys: (0, _mine(h, ys))),
            pl.BlockSpec((D, H * Dr), lambda h, ys: (0, 0)),

            pl.BlockSpec((S, Dr), lambda h, ys: (0, 0)),
            pl.BlockSpec((HG * Dh, D), lambda h, ys: (_wo_group(h, ys), 0)),
        ],
        out_specs=pl.BlockSpec((S, D), lambda h, ys: (0, 0)),
        scratch_shapes=[
            pltpu.VMEM((S, HG * Dr), F32),
            pltpu.VMEM((2, S, HG * Dh), BF16),
            pltpu.VMEM((2, S, HG * Dh), BF16),
            pltpu.SemaphoreType.DMA((2,)),
            pltpu.SemaphoreType.DMA((2,)),
        ],
    )

    ys = lax.axis_index("y").astype(jnp.int32).reshape(1)
    return pl.pallas_call(
        body,
        grid_spec=grid_spec,
        out_shape=jax.ShapeDtypeStruct((S, D), F32),
        compiler_params=pltpu.CompilerParams(
            collective_id=1,
            vmem_limit_bytes=100 * 1024 * 1024,
        ),
    )(ys, K, V, x2, Wq, Wqr, Kr, Wo)


def kernel(x, Wdkv, Wuk, Wuv, Wq, Wqr, Wkr, Wo):
    x2 = x.reshape(S, D)
    K, V, Kr = _kv_exchange(x2, Wdkv, Wuk, Wuv, Wkr)
    out = _attn_out(K, V, x2, Wq, Wqr, Kr, Wo)
    return out.reshape(B, S, D)
